# baseline (speedup 1.0000x reference)
import functools

import jax
import jax.numpy as jnp
from jax import lax
from jax.experimental import pallas as pl
from jax.experimental.pallas import tpu as pltpu
from jax.experimental.pallas import tpu_sc as plsc

NUM_CORES = 2
NUM_SUBCORES = 16
LANES = 16
NUM_WORKERS = NUM_CORES * NUM_SUBCORES

BATCH = 16384
DIM = 128
PER_WORKER = BATCH // NUM_WORKERS
CHUNK = 128
NUM_CHUNKS = PER_WORKER // CHUNK
GROUPS = CHUNK // LANES
SUB = 2
SUB_ROWS = CHUNK // SUB
SUB_GROUPS = GROUPS // SUB


PART_STRIDE = LANES + 1


NBUF = 3


def _body(users_hbm, items_hbm, utab_hbm, itab_hbm, out_hbm,
          uidx_v, iidx_v, urows0, irows0, urows1, irows1, urows2, irows2,
          part_a, part_b, out_v, semi, semo,
          sem00, sem01, sem10, sem11, sem20, sem21):
    wid = lax.axis_index("s") * NUM_CORES + lax.axis_index("c")
    base = wid * PER_WORKER

    ciu = pltpu.async_copy(users_hbm.at[pl.ds(base, PER_WORKER)], uidx_v, semi)
    cii = pltpu.async_copy(items_hbm.at[pl.ds(base, PER_WORKER)], iidx_v, semi)
    ciu.wait()
    cii.wait()

    lane = lax.iota(jnp.int32, LANES)
    lane17 = lane * PART_STRIDE
    bufs = [(urows0, irows0, (sem00, sem01)),
            (urows1, irows1, (sem10, sem11)),
            (urows2, irows2, (sem20, sem21))]

    def start(c):
        ub, ib, sems = bufs[c % NBUF]
        pairs = []
        for s in range(SUB):
            lo = s * SUB_ROWS
            cu = pltpu.async_copy(
                utab_hbm.at[uidx_v.at[pl.ds(c * CHUNK + lo, SUB_ROWS)]],
                ub.at[pl.ds(lo, SUB_ROWS)], sems[s])
            ci = pltpu.async_copy(
                itab_hbm.at[iidx_v.at[pl.ds(c * CHUNK + lo, SUB_ROWS)]],
                ib.at[pl.ds(lo, SUB_ROWS)], sems[s])
            pairs.append((cu, ci))
        return pairs

    pending = [start(c) for c in range(NBUF)]

    def load_pair(ub, ib, row):
        us = [ub[row, pl.ds(k * LANES, LANES)] for k in range(DIM // LANES)]
        vs = [ib[row, pl.ds(k * LANES, LANES)] for k in range(DIM // LANES)]
        return us, vs

    def reduce_pair(loaded):
        us, vs = loaded
        prods = [u * v for u, v in zip(us, vs)]
        while len(prods) > 1:
            prods = [a + b for a, b in zip(prods[::2], prods[1::2])]
        return prods[0]

    out_copies = []
    for c in range(NUM_CHUNKS):
        ub, ib, _ = bufs[c % NBUF]
        for s in range(SUB):
            cu, ci = pending[c % NBUF][s]
            cu.wait()
            ci.wait()

            def g_body(g, _, ub=ub, ib=ib, c=c, s=s):
                base_row = s * SUB_ROWS + g * LANES
                cur = load_pair(ub, ib, base_row)
                for j in range(LANES):
                    nxt = (load_pair(ub, ib, base_row + j + 1)
                           if j + 1 < LANES else None)
                    part_a[pl.ds(j * PART_STRIDE, LANES)] = reduce_pair(cur)
                    cur = nxt
                cols = [plsc.load_gather(part_a, [lane17 + d])
                        for d in range(LANES)]
                while len(cols) > 1:
                    cols = [a + b for a, b in zip(cols[::2], cols[1::2])]
                rating = 1.0 / (1.0 + jnp.exp(-cols[0]))
                plsc.store_scatter(
                    out_v, [c * CHUNK + base_row + lane], rating)
                return 0

            lax.fori_loop(0, SUB_GROUPS, g_body, 0)
        if c + NBUF < NUM_CHUNKS:
            pending[c % NBUF] = start(c + NBUF)
        out_copies.append(pltpu.async_copy(
            out_v.at[pl.ds(c * CHUNK, CHUNK)],
            out_hbm.at[pl.ds(base + c * CHUNK, CHUNK)], semo))
    for cp in out_copies:
        cp.wait()


@jax.jit
def kernel(users, items, user_table, item_table):
    mesh = plsc.VectorSubcoreMesh(
        core_axis_name="c", subcore_axis_name="s",
        num_cores=NUM_CORES, num_subcores=NUM_SUBCORES)
    run = pl.kernel(
        _body,
        out_type=jax.ShapeDtypeStruct((BATCH,), jnp.float32),
        mesh=mesh,
        compiler_params=pltpu.CompilerParams(
            needs_layout_passes=False, disable_bounds_checks=True),
        scratch_types=[
            pltpu.VMEM((PER_WORKER,), jnp.int32),
            pltpu.VMEM((PER_WORKER,), jnp.int32),
            pltpu.VMEM((CHUNK, DIM), jnp.float32),
            pltpu.VMEM((CHUNK, DIM), jnp.float32),
            pltpu.VMEM((CHUNK, DIM), jnp.float32),
            pltpu.VMEM((CHUNK, DIM), jnp.float32),
            pltpu.VMEM((CHUNK, DIM), jnp.float32),
            pltpu.VMEM((CHUNK, DIM), jnp.float32),
            pltpu.VMEM((LANES * PART_STRIDE,), jnp.float32),
            pltpu.VMEM((LANES * PART_STRIDE,), jnp.float32),
            pltpu.VMEM((PER_WORKER,), jnp.float32),
            pltpu.SemaphoreType.DMA,
            pltpu.SemaphoreType.DMA,
            pltpu.SemaphoreType.DMA,
            pltpu.SemaphoreType.DMA,
            pltpu.SemaphoreType.DMA,
            pltpu.SemaphoreType.DMA,
            pltpu.SemaphoreType.DMA,
            pltpu.SemaphoreType.DMA,
        ],
    )
    return run(users.astype(jnp.int32), items.astype(jnp.int32),
               user_table, item_table)


# speedup vs baseline: 1.0750x; 1.0750x over previous
import functools

import jax
import jax.numpy as jnp
from jax import lax
from jax.experimental import pallas as pl
from jax.experimental.pallas import tpu as pltpu
from jax.experimental.pallas import tpu_sc as plsc

NUM_CORES = 2
NUM_SUBCORES = 16
LANES = 16
NUM_WORKERS = NUM_CORES * NUM_SUBCORES

BATCH = 16384
DIM = 128
PER_WORKER = BATCH // NUM_WORKERS
CHUNK = 128
NUM_CHUNKS = PER_WORKER // CHUNK
GROUPS = CHUNK // LANES


PART_STRIDE = LANES + 1


NBUF = 3


def _body(users_hbm, items_hbm, utab_hbm, itab_hbm, out_hbm,
          uidx_v, iidx_v, urows0, irows0, urows1, irows1, urows2, irows2,
          part_a, out_v, semi, sem0, sem1, sem2):
    wid = lax.axis_index("s") * NUM_CORES + lax.axis_index("c")
    base = wid * PER_WORKER

    c0u = pltpu.async_copy(
        users_hbm.at[pl.ds(base, CHUNK)], uidx_v.at[pl.ds(0, CHUNK)], semi)
    c0i = pltpu.async_copy(
        items_hbm.at[pl.ds(base, CHUNK)], iidx_v.at[pl.ds(0, CHUNK)], semi)
    rest = PER_WORKER - CHUNK
    c1u = pltpu.async_copy(users_hbm.at[pl.ds(base + CHUNK, rest)],
                           uidx_v.at[pl.ds(CHUNK, rest)], semi)
    c1i = pltpu.async_copy(items_hbm.at[pl.ds(base + CHUNK, rest)],
                           iidx_v.at[pl.ds(CHUNK, rest)], semi)
    c0u.wait()
    c0i.wait()

    lane = lax.iota(jnp.int32, LANES)
    lane17 = lane * PART_STRIDE
    bufs = [(urows0, irows0, sem0), (urows1, irows1, sem1),
            (urows2, irows2, sem2)]

    def start(c):
        ub, ib, sem = bufs[c % NBUF]
        cu = pltpu.async_copy(
            utab_hbm.at[uidx_v.at[pl.ds(c * CHUNK, CHUNK)]], ub, sem)
        ci = pltpu.async_copy(
            itab_hbm.at[iidx_v.at[pl.ds(c * CHUNK, CHUNK)]], ib, sem)
        return cu, ci

    pending = [start(0)]
    c1u.wait()
    c1i.wait()
    pending += [start(c) for c in range(1, NBUF)]

    def load_pair(ub, ib, row):
        us = [ub[row, pl.ds(k * LANES, LANES)] for k in range(DIM // LANES)]
        vs = [ib[row, pl.ds(k * LANES, LANES)] for k in range(DIM // LANES)]
        return us, vs

    def reduce_pair(loaded):
        us, vs = loaded
        prods = [u * v for u, v in zip(us, vs)]
        while len(prods) > 1:
            prods = [a + b for a, b in zip(prods[::2], prods[1::2])]
        return prods[0]

    for c in range(NUM_CHUNKS):
        cu, ci = pending[c % NBUF]
        cu.wait()
        ci.wait()
        ub, ib, _ = bufs[c % NBUF]

        def g_body(g, _, ub=ub, ib=ib, c=c):
            base_row = g * LANES
            cur = load_pair(ub, ib, base_row)
            for j in range(LANES):
                nxt = (load_pair(ub, ib, base_row + j + 1)
                       if j + 1 < LANES else None)
                part_a[pl.ds(j * PART_STRIDE, LANES)] = reduce_pair(cur)
                cur = nxt
            cols = [plsc.load_gather(part_a, [lane17 + d])
                    for d in range(LANES)]
            while len(cols) > 1:
                cols = [a + b for a, b in zip(cols[::2], cols[1::2])]
            rating = 1.0 / (1.0 + jnp.exp(-cols[0]))
            plsc.store_scatter(out_v, [c * CHUNK + base_row + lane], rating)
            return 0

        lax.fori_loop(0, GROUPS, g_body, 0)
        if c + NBUF < NUM_CHUNKS:
            pending[c % NBUF] = start(c + NBUF)

    pltpu.sync_copy(out_v, out_hbm.at[pl.ds(base, PER_WORKER)])


@jax.jit
def kernel(users, items, user_table, item_table):
    mesh = plsc.VectorSubcoreMesh(
        core_axis_name="c", subcore_axis_name="s",
        num_cores=NUM_CORES, num_subcores=NUM_SUBCORES)
    run = pl.kernel(
        _body,
        out_type=jax.ShapeDtypeStruct((BATCH,), jnp.float32),
        mesh=mesh,
        compiler_params=pltpu.CompilerParams(
            needs_layout_passes=False, disable_bounds_checks=True),
        scratch_types=[
            pltpu.VMEM((PER_WORKER,), jnp.int32),
            pltpu.VMEM((PER_WORKER,), jnp.int32),
            pltpu.VMEM((CHUNK, DIM), jnp.float32),
            pltpu.VMEM((CHUNK, DIM), jnp.float32),
            pltpu.VMEM((CHUNK, DIM), jnp.float32),
            pltpu.VMEM((CHUNK, DIM), jnp.float32),
            pltpu.VMEM((CHUNK, DIM), jnp.float32),
            pltpu.VMEM((CHUNK, DIM), jnp.float32),
            pltpu.VMEM((LANES * PART_STRIDE,), jnp.float32),
            pltpu.VMEM((PER_WORKER,), jnp.float32),
            pltpu.SemaphoreType.DMA,
            pltpu.SemaphoreType.DMA,
            pltpu.SemaphoreType.DMA,
            pltpu.SemaphoreType.DMA,
        ],
    )
    return run(users.astype(jnp.int32), items.astype(jnp.int32),
               user_table, item_table)


# speedup vs baseline: 1.0754x; 1.0004x over previous
import functools

import jax
import jax.numpy as jnp
from jax import lax
from jax.experimental import pallas as pl
from jax.experimental.pallas import tpu as pltpu
from jax.experimental.pallas import tpu_sc as plsc

NUM_CORES = 2
NUM_SUBCORES = 16
LANES = 16
NUM_WORKERS = NUM_CORES * NUM_SUBCORES

BATCH = 16384
DIM = 128
PER_WORKER = BATCH // NUM_WORKERS
CHUNK = 128
NUM_CHUNKS = PER_WORKER // CHUNK
GROUPS = CHUNK // LANES


PART_STRIDE = LANES + 1


NBUF = 3


def _body(users_hbm, items_hbm, utab_hbm, itab_hbm, out_hbm,
          uidx_v, iidx_v, urows0, irows0, urows1, irows1, urows2, irows2,
          part_a, out_v, semi, semo, sem0, sem1, sem2):
    wid = lax.axis_index("s") * NUM_CORES + lax.axis_index("c")
    base = wid * PER_WORKER

    c0u = pltpu.async_copy(
        users_hbm.at[pl.ds(base, CHUNK)], uidx_v.at[pl.ds(0, CHUNK)], semi)
    c0i = pltpu.async_copy(
        items_hbm.at[pl.ds(base, CHUNK)], iidx_v.at[pl.ds(0, CHUNK)], semi)
    rest = PER_WORKER - CHUNK
    c1u = pltpu.async_copy(users_hbm.at[pl.ds(base + CHUNK, rest)],
                           uidx_v.at[pl.ds(CHUNK, rest)], semi)
    c1i = pltpu.async_copy(items_hbm.at[pl.ds(base + CHUNK, rest)],
                           iidx_v.at[pl.ds(CHUNK, rest)], semi)
    c0u.wait()
    c0i.wait()

    lane = lax.iota(jnp.int32, LANES)
    lane17 = lane * PART_STRIDE
    bufs = [(urows0, irows0, sem0), (urows1, irows1, sem1),
            (urows2, irows2, sem2)]

    def start(c):
        ub, ib, sem = bufs[c % NBUF]
        cu = pltpu.async_copy(
            utab_hbm.at[uidx_v.at[pl.ds(c * CHUNK, CHUNK)]], ub, sem)
        ci = pltpu.async_copy(
            itab_hbm.at[iidx_v.at[pl.ds(c * CHUNK, CHUNK)]], ib, sem)
        return cu, ci

    pending = [start(0)]
    c1u.wait()
    c1i.wait()
    pending += [start(c) for c in range(1, NBUF)]

    def load_pair(ub, ib, row):
        us = [ub[row, pl.ds(k * LANES, LANES)] for k in range(DIM // LANES)]
        vs = [ib[row, pl.ds(k * LANES, LANES)] for k in range(DIM // LANES)]
        return us, vs

    def reduce_pair(loaded):
        us, vs = loaded
        prods = [u * v for u, v in zip(us, vs)]
        while len(prods) > 1:
            prods = [a + b for a, b in zip(prods[::2], prods[1::2])]
        return prods[0]

    out_copies = []
    for c in range(NUM_CHUNKS):
        cu, ci = pending[c % NBUF]
        cu.wait()
        ci.wait()
        ub, ib, _ = bufs[c % NBUF]

        def g_body(g, _, ub=ub, ib=ib, c=c):
            base_row = g * LANES
            cur = load_pair(ub, ib, base_row)
            for j in range(LANES):
                nxt = (load_pair(ub, ib, base_row + j + 1)
                       if j + 1 < LANES else None)
                part_a[pl.ds(j * PART_STRIDE, LANES)] = reduce_pair(cur)
                cur = nxt
            cols = [plsc.load_gather(part_a, [lane17 + d])
                    for d in range(LANES)]
            while len(cols) > 1:
                cols = [a + b for a, b in zip(cols[::2], cols[1::2])]
            rating = 1.0 / (1.0 + jnp.exp(-cols[0]))
            plsc.store_scatter(out_v, [c * CHUNK + base_row + lane], rating)
            return 0

        lax.fori_loop(0, GROUPS, g_body, 0)
        if c + NBUF < NUM_CHUNKS:
            pending[c % NBUF] = start(c + NBUF)
        out_copies.append(pltpu.async_copy(
            out_v.at[pl.ds(c * CHUNK, CHUNK)],
            out_hbm.at[pl.ds(base + c * CHUNK, CHUNK)], semo))

    for cp in out_copies:
        cp.wait()


@jax.jit
def kernel(users, items, user_table, item_table):
    mesh = plsc.VectorSubcoreMesh(
        core_axis_name="c", subcore_axis_name="s",
        num_cores=NUM_CORES, num_subcores=NUM_SUBCORES)
    run = pl.kernel(
        _body,
        out_type=jax.ShapeDtypeStruct((BATCH,), jnp.float32),
        mesh=mesh,
        compiler_params=pltpu.CompilerParams(
            needs_layout_passes=False, disable_bounds_checks=True),
        scratch_types=[
            pltpu.VMEM((PER_WORKER,), jnp.int32),
            pltpu.VMEM((PER_WORKER,), jnp.int32),
            pltpu.VMEM((CHUNK, DIM), jnp.float32),
            pltpu.VMEM((CHUNK, DIM), jnp.float32),
            pltpu.VMEM((CHUNK, DIM), jnp.float32),
            pltpu.VMEM((CHUNK, DIM), jnp.float32),
            pltpu.VMEM((CHUNK, DIM), jnp.float32),
            pltpu.VMEM((CHUNK, DIM), jnp.float32),
            pltpu.VMEM((LANES * PART_STRIDE,), jnp.float32),
            pltpu.VMEM((PER_WORKER,), jnp.float32),
            pltpu.SemaphoreType.DMA,
            pltpu.SemaphoreType.DMA,
            pltpu.SemaphoreType.DMA,
            pltpu.SemaphoreType.DMA,
            pltpu.SemaphoreType.DMA,
        ],
    )
    return run(users.astype(jnp.int32), items.astype(jnp.int32),
               user_table, item_table)


# speedup vs baseline: 1.0927x; 1.0161x over previous
import functools

import jax
import jax.numpy as jnp
from jax import lax
from jax.experimental import pallas as pl
from jax.experimental.pallas import tpu as pltpu
from jax.experimental.pallas import tpu_sc as plsc

NUM_CORES = 2
NUM_SUBCORES = 16
LANES = 16
NUM_WORKERS = NUM_CORES * NUM_SUBCORES

BATCH = 16384
DIM = 128
PER_WORKER = BATCH // NUM_WORKERS
CHUNK = 128
NUM_CHUNKS = PER_WORKER // CHUNK
GROUPS = CHUNK // LANES


PART_STRIDE = LANES + 1


NBUF = 2


def _body(users_hbm, items_hbm, utab_hbm, itab_hbm, out_hbm,
          uidx_v, iidx_v, urows0, irows0, urows1, irows1,
          part_a, out_v, semi, semo, sem0, sem1):
    wid = lax.axis_index("s") * NUM_CORES + lax.axis_index("c")
    base = wid * PER_WORKER

    c0u = pltpu.async_copy(
        users_hbm.at[pl.ds(base, CHUNK)], uidx_v.at[pl.ds(0, CHUNK)], semi)
    c0i = pltpu.async_copy(
        items_hbm.at[pl.ds(base, CHUNK)], iidx_v.at[pl.ds(0, CHUNK)], semi)
    rest = PER_WORKER - CHUNK
    c1u = pltpu.async_copy(users_hbm.at[pl.ds(base + CHUNK, rest)],
                           uidx_v.at[pl.ds(CHUNK, rest)], semi)
    c1i = pltpu.async_copy(items_hbm.at[pl.ds(base + CHUNK, rest)],
                           iidx_v.at[pl.ds(CHUNK, rest)], semi)
    c0u.wait()
    c0i.wait()

    lane = lax.iota(jnp.int32, LANES)
    lane17 = lane * PART_STRIDE
    bufs = [(urows0, irows0, sem0), (urows1, irows1, sem1)]

    def start(c):
        ub, ib, sem = bufs[c % NBUF]
        cu = pltpu.async_copy(
            utab_hbm.at[uidx_v.at[pl.ds(c * CHUNK, CHUNK)]], ub, sem)
        ci = pltpu.async_copy(
            itab_hbm.at[iidx_v.at[pl.ds(c * CHUNK, CHUNK)]], ib, sem)
        return cu, ci

    pending = [start(0)]
    c1u.wait()
    c1i.wait()
    pending += [start(c) for c in range(1, NBUF)]

    def load_pair(ub, ib, row):
        us = [ub[row, pl.ds(k * LANES, LANES)] for k in range(DIM // LANES)]
        vs = [ib[row, pl.ds(k * LANES, LANES)] for k in range(DIM // LANES)]
        return us, vs

    def reduce_pair(loaded):
        us, vs = loaded
        prods = [u * v for u, v in zip(us, vs)]
        while len(prods) > 1:
            prods = [a + b for a, b in zip(prods[::2], prods[1::2])]
        return prods[0]

    out_copies = []
    for c in range(NUM_CHUNKS):
        cu, ci = pending[c % NBUF]
        cu.wait()
        ci.wait()
        ub, ib, _ = bufs[c % NBUF]

        def g_body(g, _, ub=ub, ib=ib, c=c):
            base_row = g * LANES
            cur = load_pair(ub, ib, base_row)
            for j in range(LANES):
                nxt = (load_pair(ub, ib, base_row + j + 1)
                       if j + 1 < LANES else None)
                part_a[pl.ds(j * PART_STRIDE, LANES)] = reduce_pair(cur)
                cur = nxt
            cols = [plsc.load_gather(part_a, [lane17 + d])
                    for d in range(LANES)]
            while len(cols) > 1:
                cols = [a + b for a, b in zip(cols[::2], cols[1::2])]
            rating = 1.0 / (1.0 + jnp.exp(-cols[0]))
            plsc.store_scatter(out_v, [c * CHUNK + base_row + lane], rating)
            return 0

        lax.fori_loop(0, GROUPS, g_body, 0)
        if c + NBUF < NUM_CHUNKS:
            pending[c % NBUF] = start(c + NBUF)
        out_copies.append(pltpu.async_copy(
            out_v.at[pl.ds(c * CHUNK, CHUNK)],
            out_hbm.at[pl.ds(base + c * CHUNK, CHUNK)], semo))

    for cp in out_copies:
        cp.wait()


@jax.jit
def kernel(users, items, user_table, item_table):
    mesh = plsc.VectorSubcoreMesh(
        core_axis_name="c", subcore_axis_name="s",
        num_cores=NUM_CORES, num_subcores=NUM_SUBCORES)
    run = pl.kernel(
        _body,
        out_type=jax.ShapeDtypeStruct((BATCH,), jnp.float32),
        mesh=mesh,
        compiler_params=pltpu.CompilerParams(
            needs_layout_passes=False, disable_bounds_checks=True),
        scratch_types=[
            pltpu.VMEM((PER_WORKER,), jnp.int32),
            pltpu.VMEM((PER_WORKER,), jnp.int32),
            pltpu.VMEM((CHUNK, DIM), jnp.float32),
            pltpu.VMEM((CHUNK, DIM), jnp.float32),
            pltpu.VMEM((CHUNK, DIM), jnp.float32),
            pltpu.VMEM((CHUNK, DIM), jnp.float32),
            pltpu.VMEM((LANES * PART_STRIDE,), jnp.float32),
            pltpu.VMEM((PER_WORKER,), jnp.float32),
            pltpu.SemaphoreType.DMA,
            pltpu.SemaphoreType.DMA,
            pltpu.SemaphoreType.DMA,
            pltpu.SemaphoreType.DMA,
        ],
    )
    return run(users.astype(jnp.int32), items.astype(jnp.int32),
               user_table, item_table)


# speedup vs baseline: 1.0978x; 1.0047x over previous
import functools

import jax
import jax.numpy as jnp
from jax import lax
from jax.experimental import pallas as pl
from jax.experimental.pallas import tpu as pltpu
from jax.experimental.pallas import tpu_sc as plsc

NUM_CORES = 2
NUM_SUBCORES = 16
LANES = 16
NUM_WORKERS = NUM_CORES * NUM_SUBCORES

BATCH = 16384
DIM = 128
PER_WORKER = BATCH // NUM_WORKERS
CHUNK = 128
NUM_CHUNKS = PER_WORKER // CHUNK
GROUPS = CHUNK // LANES


PART_STRIDE = LANES + 1


NBUF = 2


def _body(users_hbm, items_hbm, utab_hbm, itab_hbm, out_hbm,
          uidx_v, iidx_v, urows0, irows0, urows1, irows1,
          part_a, out_v, semi, semo, sem0, sem1):
    wid = lax.axis_index("s") * NUM_CORES + lax.axis_index("c")
    base = wid * PER_WORKER

    c0u = pltpu.async_copy(
        users_hbm.at[pl.ds(base, CHUNK)], uidx_v.at[pl.ds(0, CHUNK)], semi)
    c0i = pltpu.async_copy(
        items_hbm.at[pl.ds(base, CHUNK)], iidx_v.at[pl.ds(0, CHUNK)], semi)
    rest = PER_WORKER - CHUNK
    c1u = pltpu.async_copy(users_hbm.at[pl.ds(base + CHUNK, rest)],
                           uidx_v.at[pl.ds(CHUNK, rest)], semi)
    c1i = pltpu.async_copy(items_hbm.at[pl.ds(base + CHUNK, rest)],
                           iidx_v.at[pl.ds(CHUNK, rest)], semi)
    c0u.wait()
    c0i.wait()

    lane = lax.iota(jnp.int32, LANES)
    lane17 = lane * PART_STRIDE
    bufs = [(urows0, irows0, sem0), (urows1, irows1, sem1)]

    def start(c):
        ub, ib, sem = bufs[c % NBUF]
        cu = pltpu.async_copy(
            utab_hbm.at[uidx_v.at[pl.ds(c * CHUNK, CHUNK)]], ub, sem)
        ci = pltpu.async_copy(
            itab_hbm.at[iidx_v.at[pl.ds(c * CHUNK, CHUNK)]], ib, sem)
        return cu, ci

    def start0():
        ub, ib, sem = bufs[0]
        ps = []
        for s in range(2):
            lo = s * (CHUNK // 2)
            cu = pltpu.async_copy(
                utab_hbm.at[uidx_v.at[pl.ds(lo, CHUNK // 2)]],
                ub.at[pl.ds(lo, CHUNK // 2)], sem)
            ci = pltpu.async_copy(
                itab_hbm.at[iidx_v.at[pl.ds(lo, CHUNK // 2)]],
                ib.at[pl.ds(lo, CHUNK // 2)], sem)
            ps.append((cu, ci))
        return ps

    pending0 = start0()
    c1u.wait()
    c1i.wait()
    pending = [None] + [start(c) for c in range(1, NBUF)]

    def load_pair(ub, ib, row):
        us = [ub[row, pl.ds(k * LANES, LANES)] for k in range(DIM // LANES)]
        vs = [ib[row, pl.ds(k * LANES, LANES)] for k in range(DIM // LANES)]
        return us, vs

    def reduce_pair(loaded):
        us, vs = loaded
        prods = [u * v for u, v in zip(us, vs)]
        while len(prods) > 1:
            prods = [a + b for a, b in zip(prods[::2], prods[1::2])]
        return prods[0]

    out_copies = []
    for c in range(NUM_CHUNKS):
        ub, ib, _ = bufs[c % NBUF]

        def g_body(g, _, ub=ub, ib=ib, c=c):
            base_row = g * LANES
            cur = load_pair(ub, ib, base_row)
            for j in range(LANES):
                nxt = (load_pair(ub, ib, base_row + j + 1)
                       if j + 1 < LANES else None)
                part_a[pl.ds(j * PART_STRIDE, LANES)] = reduce_pair(cur)
                cur = nxt
            cols = [plsc.load_gather(part_a, [lane17 + d])
                    for d in range(LANES)]
            while len(cols) > 1:
                cols = [a + b for a, b in zip(cols[::2], cols[1::2])]
            rating = 1.0 / (1.0 + jnp.exp(-cols[0]))
            plsc.store_scatter(out_v, [c * CHUNK + base_row + lane], rating)
            return 0

        if c == 0:
            for s in range(2):
                cu, ci = pending0[s]
                cu.wait()
                ci.wait()
                lax.fori_loop(s * (GROUPS // 2), (s + 1) * (GROUPS // 2),
                              g_body, 0)
        else:
            cu, ci = pending[c % NBUF]
            cu.wait()
            ci.wait()
            lax.fori_loop(0, GROUPS, g_body, 0)
        if c + NBUF < NUM_CHUNKS:
            pending[c % NBUF] = start(c + NBUF)
        out_copies.append(pltpu.async_copy(
            out_v.at[pl.ds(c * CHUNK, CHUNK)],
            out_hbm.at[pl.ds(base + c * CHUNK, CHUNK)], semo))

    for cp in out_copies:
        cp.wait()


@jax.jit
def kernel(users, items, user_table, item_table):
    mesh = plsc.VectorSubcoreMesh(
        core_axis_name="c", subcore_axis_name="s",
        num_cores=NUM_CORES, num_subcores=NUM_SUBCORES)
    run = pl.kernel(
        _body,
        out_type=jax.ShapeDtypeStruct((BATCH,), jnp.float32),
        mesh=mesh,
        compiler_params=pltpu.CompilerParams(
            needs_layout_passes=False, disable_bounds_checks=True),
        scratch_types=[
            pltpu.VMEM((PER_WORKER,), jnp.int32),
            pltpu.VMEM((PER_WORKER,), jnp.int32),
            pltpu.VMEM((CHUNK, DIM), jnp.float32),
            pltpu.VMEM((CHUNK, DIM), jnp.float32),
            pltpu.VMEM((CHUNK, DIM), jnp.float32),
            pltpu.VMEM((CHUNK, DIM), jnp.float32),
            pltpu.VMEM((LANES * PART_STRIDE,), jnp.float32),
            pltpu.VMEM((PER_WORKER,), jnp.float32),
            pltpu.SemaphoreType.DMA,
            pltpu.SemaphoreType.DMA,
            pltpu.SemaphoreType.DMA,
            pltpu.SemaphoreType.DMA,
        ],
    )
    return run(users.astype(jnp.int32), items.astype(jnp.int32),
               user_table, item_table)
